# baseline (device time: 101138 ns/iter reference)
import jax
import jax.numpy as jnp
from jax import lax
from jax.experimental import pallas as pl
from jax.experimental.pallas import tpu as pltpu

N_DEV = 4


def kernel(x, w_mat, scale_x, scale_w):
    if x.dtype != jnp.float8_e5m2:
        x = x.astype(jnp.float8_e5m2)
    scale_x = scale_x.astype(jnp.float32)
    scale_w = scale_w.astype(jnp.float32)

    m_per, k = x.shape
    _, n_per = w_mat.shape
    half = m_per // 2

    def body(x_ref, w_ref, sx_ref, sw_ref, out_ref,
             wq, rl_buf, rr_buf, hl_buf, hr_buf, send_sems, recv_sems):
        my = lax.axis_index("i")
        left = lax.rem(my + (N_DEV - 1), N_DEV)
        right = lax.rem(my + 1, N_DEV)

        barrier_sem = pltpu.get_barrier_semaphore()
        for nbr in (left, right):
            pl.semaphore_signal(
                barrier_sem, inc=1,
                device_id=(nbr,), device_id_type=pl.DeviceIdType.MESH,
            )
        pl.semaphore_wait(barrier_sem, 2)

        s1r = pltpu.make_async_remote_copy(
            src_ref=x_ref, dst_ref=rl_buf,
            send_sem=send_sems.at[0], recv_sem=recv_sems.at[0],
            device_id=(right,), device_id_type=pl.DeviceIdType.MESH,
        )
        s1l = pltpu.make_async_remote_copy(
            src_ref=x_ref, dst_ref=rr_buf,
            send_sem=send_sems.at[1], recv_sem=recv_sems.at[1],
            device_id=(left,), device_id_type=pl.DeviceIdType.MESH,
        )
        s1r.start()
        s1l.start()

        wq[...] = w_ref[...].astype(jnp.float8_e5m2)

        scale = sx_ref[0] * sw_ref[0]

        def gemm_store(chunk, row_start):
            acc = jnp.dot(chunk, wq[...], preferred_element_type=jnp.float32)
            y = acc * scale
            z = jnp.clip(y, -60.0, 60.0)
            out_ref[pl.ds(row_start, chunk.shape[0]), :] = y * (
                1.0 / (1.0 + jnp.exp(-z))
            )

        gemm_store(x_ref[...], my * m_per)

        s1r.wait_recv()
        s2r = pltpu.make_async_remote_copy(
            src_ref=rl_buf.at[pl.ds(0, half)], dst_ref=hl_buf,
            send_sem=send_sems.at[2], recv_sem=recv_sems.at[2],
            device_id=(right,), device_id_type=pl.DeviceIdType.MESH,
        )
        s2r.start()
        gemm_store(rl_buf[...], left * m_per)

        s1l.wait_recv()
        s2l = pltpu.make_async_remote_copy(
            src_ref=rr_buf.at[pl.ds(half, half)], dst_ref=hr_buf,
            send_sem=send_sems.at[3], recv_sem=recv_sems.at[3],
            device_id=(left,), device_id_type=pl.DeviceIdType.MESH,
        )
        s2l.start()
        gemm_store(rr_buf[...], right * m_per)

        opp_row = lax.rem(my + 2, N_DEV) * m_per
        s2r.wait_recv()
        gemm_store(hl_buf[...], opp_row)
        s2l.wait_recv()
        gemm_store(hr_buf[...], opp_row + half)

        s1r.wait_send()
        s1l.wait_send()
        s2r.wait_send()
        s2l.wait_send()

    return pl.pallas_call(
        body,
        out_shape=jax.ShapeDtypeStruct((N_DEV * m_per, n_per), jnp.float32),
        in_specs=[
            pl.BlockSpec(memory_space=pltpu.VMEM),
            pl.BlockSpec(memory_space=pltpu.VMEM),
            pl.BlockSpec(memory_space=pltpu.SMEM),
            pl.BlockSpec(memory_space=pltpu.SMEM),
        ],
        out_specs=pl.BlockSpec(memory_space=pltpu.VMEM),
        scratch_shapes=[
            pltpu.VMEM((k, n_per), jnp.float8_e5m2),
            pltpu.VMEM((m_per, k), jnp.float8_e5m2),
            pltpu.VMEM((m_per, k), jnp.float8_e5m2),
            pltpu.VMEM((half, k), jnp.float8_e5m2),
            pltpu.VMEM((half, k), jnp.float8_e5m2),
            pltpu.SemaphoreType.DMA((4,)),
            pltpu.SemaphoreType.DMA((4,)),
        ],
        compiler_params=pltpu.CompilerParams(
            collective_id=0, vmem_limit_bytes=64 * 1024 * 1024,
        ),
    )(x, w_mat, scale_x, scale_w)


# device time: 86715 ns/iter; 1.1663x vs baseline; 1.1663x over previous
import jax
import jax.numpy as jnp
from jax import lax
from jax.experimental import pallas as pl
from jax.experimental.pallas import tpu as pltpu

N_DEV = 4


def kernel(x, w_mat, scale_x, scale_w):
    scale_x = scale_x.astype(jnp.float32)
    scale_w = scale_w.astype(jnp.float32)

    m_per, k = x.shape
    _, n_per = w_mat.shape
    half = m_per // 2

    def body(x_ref, w_ref, sx_ref, sw_ref, out_ref,
             xf, wf, xq, wq, rl_buf, rr_buf, hl_buf, hr_buf, stage,
             send_sems, recv_sems, out_sems, in_sems):
        my = lax.axis_index("i")
        left = lax.rem(my + (N_DEV - 1), N_DEV)
        right = lax.rem(my + 1, N_DEV)

        cp_xt = pltpu.make_async_copy(
            x_ref.at[pl.ds(0, half)], xf.at[pl.ds(0, half)], in_sems.at[0])
        cp_xb = pltpu.make_async_copy(
            x_ref.at[pl.ds(half, half)], xf.at[pl.ds(half, half)], in_sems.at[1])
        cp_w = pltpu.make_async_copy(w_ref, wf, in_sems.at[2])
        cp_xt.start()
        cp_xb.start()
        cp_w.start()

        barrier_sem = pltpu.get_barrier_semaphore()
        for nbr in (left, right):
            pl.semaphore_signal(
                barrier_sem, inc=1,
                device_id=(nbr,), device_id_type=pl.DeviceIdType.MESH,
            )
        pl.semaphore_wait(barrier_sem, 2)

        s1r_a = pltpu.make_async_remote_copy(
            src_ref=xq.at[pl.ds(0, half)], dst_ref=rl_buf.at[pl.ds(0, half)],
            send_sem=send_sems.at[0], recv_sem=recv_sems.at[0],
            device_id=(right,), device_id_type=pl.DeviceIdType.MESH,
        )
        s1r_b = pltpu.make_async_remote_copy(
            src_ref=xq.at[pl.ds(half, half)], dst_ref=rl_buf.at[pl.ds(half, half)],
            send_sem=send_sems.at[1], recv_sem=recv_sems.at[1],
            device_id=(right,), device_id_type=pl.DeviceIdType.MESH,
        )
        s1l_a = pltpu.make_async_remote_copy(
            src_ref=xq.at[pl.ds(half, half)], dst_ref=rr_buf.at[pl.ds(half, half)],
            send_sem=send_sems.at[2], recv_sem=recv_sems.at[2],
            device_id=(left,), device_id_type=pl.DeviceIdType.MESH,
        )
        s1l_b = pltpu.make_async_remote_copy(
            src_ref=xq.at[pl.ds(0, half)], dst_ref=rr_buf.at[pl.ds(0, half)],
            send_sem=send_sems.at[3], recv_sem=recv_sems.at[3],
            device_id=(left,), device_id_type=pl.DeviceIdType.MESH,
        )

        cp_xt.wait()
        xq[pl.ds(0, half)] = xf[pl.ds(0, half)].astype(jnp.float8_e5m2)
        s1r_a.start()
        cp_xb.wait()
        xq[pl.ds(half, half)] = xf[pl.ds(half, half)].astype(jnp.float8_e5m2)
        s1l_a.start()
        s1r_b.start()
        s1l_b.start()

        cp_w.wait()
        wq[...] = wf[...].astype(jnp.float8_e5m2)

        scale = sx_ref[0] * sw_ref[0]
        out_copies = []

        def gemm_store(chunk, row_start, slot, rows):
            acc = jnp.dot(chunk, wq[...], preferred_element_type=jnp.float32)
            y = acc * scale
            z = jnp.clip(y, -60.0, 60.0)
            stage[slot, pl.ds(0, rows), :] = y * (1.0 / (1.0 + jnp.exp(-z)))
            cp = pltpu.make_async_copy(
                stage.at[slot, pl.ds(0, rows)],
                out_ref.at[pl.ds(row_start, rows)],
                out_sems.at[slot],
            )
            cp.start()
            out_copies.append(cp)

        gemm_store(xq[...], my * m_per, 0, m_per)

        s1r_a.wait_recv()
        s2r = pltpu.make_async_remote_copy(
            src_ref=rl_buf.at[pl.ds(0, half)], dst_ref=hl_buf,
            send_sem=send_sems.at[4], recv_sem=recv_sems.at[4],
            device_id=(right,), device_id_type=pl.DeviceIdType.MESH,
        )
        s2r.start()

        s1l_a.wait_recv()
        s2l = pltpu.make_async_remote_copy(
            src_ref=rr_buf.at[pl.ds(half, half)], dst_ref=hr_buf,
            send_sem=send_sems.at[5], recv_sem=recv_sems.at[5],
            device_id=(left,), device_id_type=pl.DeviceIdType.MESH,
        )
        s2l.start()

        s1r_b.wait_recv()
        gemm_store(rl_buf[...], left * m_per, 1, m_per)
        s1l_b.wait_recv()
        gemm_store(rr_buf[...], right * m_per, 2, m_per)

        opp_row = lax.rem(my + 2, N_DEV) * m_per
        s2r.wait_recv()
        gemm_store(hl_buf[...], opp_row, 3, half)
        s2l.wait_recv()
        gemm_store(hr_buf[...], opp_row + half, 4, half)

        for s in (s1r_a, s1r_b, s1l_a, s1l_b, s2r, s2l):
            s.wait_send()
        for cp in out_copies:
            cp.wait()

    return pl.pallas_call(
        body,
        out_shape=jax.ShapeDtypeStruct((N_DEV * m_per, n_per), jnp.float32),
        in_specs=[
            pl.BlockSpec(memory_space=pl.ANY),
            pl.BlockSpec(memory_space=pl.ANY),
            pl.BlockSpec(memory_space=pltpu.SMEM),
            pl.BlockSpec(memory_space=pltpu.SMEM),
        ],
        out_specs=pl.BlockSpec(memory_space=pl.ANY),
        scratch_shapes=[
            pltpu.VMEM((m_per, k), jnp.float32),
            pltpu.VMEM((k, n_per), jnp.float32),
            pltpu.VMEM((m_per, k), jnp.float8_e5m2),
            pltpu.VMEM((k, n_per), jnp.float8_e5m2),
            pltpu.VMEM((m_per, k), jnp.float8_e5m2),
            pltpu.VMEM((m_per, k), jnp.float8_e5m2),
            pltpu.VMEM((half, k), jnp.float8_e5m2),
            pltpu.VMEM((half, k), jnp.float8_e5m2),
            pltpu.VMEM((5, m_per, n_per), jnp.float32),
            pltpu.SemaphoreType.DMA((6,)),
            pltpu.SemaphoreType.DMA((6,)),
            pltpu.SemaphoreType.DMA((5,)),
            pltpu.SemaphoreType.DMA((3,)),
        ],
        compiler_params=pltpu.CompilerParams(
            collective_id=0, vmem_limit_bytes=100 * 1024 * 1024,
        ),
    )(x, w_mat, scale_x, scale_w)
